# baseline (speedup 1.0000x reference)
import jax
import jax.numpy as jnp
from jax import lax
from jax.experimental import pallas as pl
from jax.experimental.pallas import tpu as pltpu
from jax.experimental.pallas import tpu_sc as plsc

N_USERS = 25000
N_ITEMS = 25000
D = 64
E = 800000

NTILE = 16
EPT = E // NTILE
C = 50
S = 1000
NCH = S // C
NQ = NCH // 5
NS = EPT // S

RB = 40
NRB = N_USERS // RB
RB_ITERS = -(-NRB // NTILE)

THIRD = 1.0 / 3.0

_mesh = plsc.VectorSubcoreMesh(core_axis_name="c", subcore_axis_name="s")


def _zero_wbuf(wbuf):
    def zrow(r, c):
        for d in range(D // 16):
            wbuf[r, pl.ds(d * 16, 16)] = jnp.zeros((16,), jnp.float32)
        return c
    lax.fori_loop(0, RB, zrow, 0)


_GROUPS = ((0, range(16)), (16, range(16)), (32, range(16)),
           (34, range(14, 16)))


def _scale_chunk(p2, cs, pkb, gbuf, mbuf):
    for off, lanes in _GROUPS:
        vals16 = plsc.bitcast(pkb[p2, cs, 2, pl.ds(off, 16)], jnp.float32)
        for j in lanes:
            v = vals16[j]
            e = off + j
            for d in range(D // 16):
                sl = pl.ds(d * 16, 16)
                mbuf[e, sl] = gbuf[e, sl] * v


def _side(sid, src, pkd, gi, si, out,
          pkb, gbufs, mbufs, wbuf,
          acc, isem, gsems, ssems, base_prev):
    rbase = sid * (EPT // C)

    pltpu.async_copy(pkd.at[pl.ds(rbase, NCH)], pkb.at[0], isem)

    _zero_wbuf(wbuf)

    def zchunk(k, c):
        chunk = k * NTILE + sid

        @pl.when(chunk < NRB)
        def _():
            pltpu.sync_copy(wbuf, acc.at[pl.ds(chunk * RB, RB)])
        return c
    lax.fori_loop(0, RB_ITERS, zchunk, 0)
    plsc.subcore_barrier()

    pltpu.make_async_copy(pkd.at[pl.ds(rbase, NCH)], pkb.at[0], isem).wait()
    for i in range(5):
        pltpu.async_copy(src.at[pkb.at[0, i, gi]], gbufs[i], gsems[i])

    def sbody(s, c):
        p2 = s & 1
        q2 = 1 - p2
        nb = rbase + (s + 1) * NCH

        @pl.when(s + 1 < NS)
        def _():
            pltpu.async_copy(pkd.at[pl.ds(nb, NCH)], pkb.at[q2], isem)

        def qbody(q, c2):
            for i in range(5):
                cs = 5 * q + i
                gb = gbufs[i]
                mb = mbufs[i & 1]
                gsem = gsems[i]
                ssem = ssems[i & 1]

                pltpu.make_async_copy(src.at[pkb.at[p2, cs, gi]], gb,
                                      gsem).wait()
                if i < 2:
                    @pl.when(q > 0)
                    def _(mb=mb, cs=cs, ssem=ssem):
                        pltpu.make_async_copy(
                            mb, acc.at[pkb.at[p2, cs - 2, si]], ssem).wait()
                else:
                    pltpu.make_async_copy(
                        mb, acc.at[pkb.at[p2, cs - 2, si]], ssem).wait()
                _scale_chunk(p2, cs, pkb, gb, mb)

                @pl.when(q < NQ - 1)
                def _(gb=gb, cs=cs, gsem=gsem):
                    pltpu.async_copy(src.at[pkb.at[p2, cs + 5, gi]], gb, gsem)
                pltpu.async_copy(mb, acc.at[pkb.at[p2, cs, si]], ssem,
                                 add=True)
            return c2
        lax.fori_loop(0, NQ, qbody, 0)

        pltpu.make_async_copy(mbufs[0], acc.at[pkb.at[p2, NCH - 2, si]],
                              ssems[0]).wait()
        pltpu.make_async_copy(mbufs[1], acc.at[pkb.at[p2, NCH - 1, si]],
                              ssems[1]).wait()


        @pl.when(s + 1 < NS)
        def _():
            pltpu.make_async_copy(pkd.at[pl.ds(nb, NCH)], pkb.at[q2],
                                  isem).wait()
            for i in range(5):
                pltpu.async_copy(src.at[pkb.at[q2, i, gi]], gbufs[i],
                                 gsems[i])
        return c
    lax.fori_loop(0, NS, sbody, 0)
    plsc.subcore_barrier()

    def wchunk(k, c):
        chunk = k * NTILE + sid

        @pl.when(chunk < NRB)
        def _():
            rb = chunk * RB
            pltpu.sync_copy(acc.at[pl.ds(rb, RB)], wbuf)
            if base_prev is not None:
                bref, pref, bbuf = base_prev
                pltpu.sync_copy(bref.at[pl.ds(rb, RB)], bbuf)

                def arow(r, c2):
                    for d in range(D // 16):
                        sl = pl.ds(d * 16, 16)
                        wbuf[r, sl] = wbuf[r, sl] + bbuf[r, sl]
                    return c2
                lax.fori_loop(0, RB, arow, 0)
                pltpu.sync_copy(pref.at[pl.ds(rb, RB)], bbuf)

                def prow(r, c2):
                    for d in range(D // 16):
                        sl = pl.ds(d * 16, 16)
                        wbuf[r, sl] = (wbuf[r, sl] + bbuf[r, sl]) * THIRD
                    return c2
                lax.fori_loop(0, RB, prow, 0)
            pltpu.sync_copy(wbuf, out.at[pl.ds(rb, RB)])
        return c
    lax.fori_loop(0, RB_ITERS, wchunk, 0)


def _layer1_body(user_w, item_w, pkd, out_zu, out_zi,
                 pkb, g0, g1, g2, g3, g4, m0, m1, acc,
                 isem, gs0, gs1, gs2, gs3, gs4, ss0, ss1):
    cid = lax.axis_index("c")
    sid = lax.axis_index("s")
    gbufs = (g0, g1, g2, g3, g4)
    mbufs = (m0, m1)
    gsems = (gs0, gs1, gs2, gs3, gs4)
    ssems = (ss0, ss1)
    wbuf = g0.at[pl.ds(0, RB)]

    @pl.when(cid == 0)
    def _():
        _side(sid, item_w, pkd, 1, 0, out_zu,
              pkb, gbufs, mbufs, wbuf, acc,
              isem, gsems, ssems, None)

    @pl.when(cid == 1)
    def _():
        _side(sid, user_w, pkd, 0, 1, out_zi,
              pkb, gbufs, mbufs, wbuf, acc,
              isem, gsems, ssems, None)


def _layer2_body(user_w, item_w, zu1, zi1, pkd, out_u, out_i,
                 pkb, g0, g1, g2, g3, g4, m0, m1, acc,
                 isem, gs0, gs1, gs2, gs3, gs4, ss0, ss1):
    cid = lax.axis_index("c")
    sid = lax.axis_index("s")
    gbufs = (g0, g1, g2, g3, g4)
    mbufs = (m0, m1)
    gsems = (gs0, gs1, gs2, gs3, gs4)
    ssems = (ss0, ss1)
    wbuf = g0.at[pl.ds(0, RB)]
    bbuf = g1.at[pl.ds(0, RB)]

    @pl.when(cid == 0)
    def _():
        _side(sid, zi1, pkd, 1, 0, out_u,
              pkb, gbufs, mbufs, wbuf, acc,
              isem, gsems, ssems, (user_w, zu1, bbuf))

    @pl.when(cid == 1)
    def _():
        _side(sid, zu1, pkd, 0, 1, out_i,
              pkb, gbufs, mbufs, wbuf, acc,
              isem, gsems, ssems, (item_w, zi1, bbuf))


_f32 = jnp.float32
_emb = jax.ShapeDtypeStruct((N_USERS, D), _f32)

_common_scratch = [
    pltpu.VMEM((2, NCH, 3, C), jnp.int32),
    pltpu.VMEM((C, D), _f32),
    pltpu.VMEM((C, D), _f32),
    pltpu.VMEM((C, D), _f32),
    pltpu.VMEM((C, D), _f32),
    pltpu.VMEM((C, D), _f32),
    pltpu.VMEM((C, D), _f32),
    pltpu.VMEM((C, D), _f32),
]

_sems = [pltpu.SemaphoreType.DMA] * 8

_params = pltpu.CompilerParams(use_tc_tiling_on_sc=False,
                               needs_layout_passes=False)

_layer1 = pl.kernel(
    _layer1_body,
    out_type=(_emb, _emb),
    mesh=_mesh,
    compiler_params=_params,
    scratch_types=_common_scratch + [
        pltpu.VMEM_SHARED((N_USERS, D), _f32),
    ] + _sems,
)

_layer2 = pl.kernel(
    _layer2_body,
    out_type=(_emb, _emb),
    mesh=_mesh,
    compiler_params=_params,
    scratch_types=_common_scratch + [
        pltpu.VMEM_SHARED((N_USERS, D), _f32),
    ] + _sems,
)


def kernel(user_w, item_w, adj_rows, adj_cols, adj_vals,
           image_item_embeds, text_item_embeds,
           image_user_embeds, text_user_embeds):
    pkd = jnp.stack([adj_rows.reshape(E // C, C),
                     adj_cols.reshape(E // C, C),
                     lax.bitcast_convert_type(adj_vals,
                                              jnp.int32).reshape(E // C, C)],
                    axis=1)
    zu1, zi1 = _layer1(user_w, item_w, pkd)
    return _layer2(user_w, item_w, zu1, zi1, pkd)
